# baseline (speedup 1.0000x reference)
import functools

import jax
import jax.numpy as jnp
from jax import lax
from jax.experimental import pallas as pl
from jax.experimental.pallas import tpu as pltpu
from jax.experimental.pallas import tpu_sc as plsc

N = 10000
E = 320000
IN_DIM = 128
D = 64
AVG_LOG = 3.4965075614664802
CAT_DIM = 10 * D

NT = 32
NB = 320
NPAD = NT * NB
NBP = NB + 8
CAP = 12288
CH = 128
CHS = 2000

_NEG = -3.0e38
_POS = 3.0e38

_mesh = plsc.VectorSubcoreMesh(core_axis_name="c", subcore_axis_name="s")
_sc_params = pltpu.CompilerParams(needs_layout_passes=False,
                                  use_tc_tiling_on_sc=False)


def _lane_bcast(v, j):
    idx = jnp.full((16, 1), j, jnp.int32)
    return lax.gather(
        v, idx,
        lax.GatherDimensionNumbers(
            offset_dims=(), collapsed_slice_dims=(0,), start_index_map=(0,)),
        (1,), mode=lax.GatherScatterMode.PROMISE_IN_BOUNDS)


NC_SETUP = E // CHS


@functools.partial(
    pl.kernel,
    out_type=(
        jax.ShapeDtypeStruct((NT, CAP), jnp.int32),
        jax.ShapeDtypeStruct((NT, CAP), jnp.int32),
        jax.ShapeDtypeStruct((NT, 16), jnp.int32),
        jax.ShapeDtypeStruct((NPAD,), jnp.float32),
    ),
    mesh=_mesh,
    compiler_params=_sc_params,
    scratch_types=[
        pltpu.VMEM((2, CHS), jnp.int32),
        pltpu.VMEM((2, CHS), jnp.int32),
        pltpu.VMEM((CAP,), jnp.int32),
        pltpu.VMEM((CAP,), jnp.int32),
        pltpu.VMEM((16, NB), jnp.float32),
        pltpu.VMEM((NB,), jnp.float32),
        pltpu.VMEM((16,), jnp.int32),
        pltpu.SemaphoreType.DMA,
        pltpu.SemaphoreType.DMA,
    ],
)
def _sc_setup(g_hbm, srcl_hbm, ldstl_hbm, cnt_hbm, deg_hbm,
              gb0, gb1, srco, ldsto, hist, degb, cntb, sem0, sem1):
    cid = lax.axis_index("c")
    sid = lax.axis_index("s")
    wid = sid * 2 + cid
    lo = wid * NB
    iota = lax.iota(jnp.int32, 16)
    zero16f = jnp.zeros((16,), jnp.float32)
    zero16i = jnp.zeros((16,), jnp.int32)

    def start(ci, buf, sem):
        pltpu.async_copy(g_hbm.at[:, pl.ds(ci * CHS, CHS)], buf, sem)

    def wait(buf, sem):
        pltpu.make_async_copy(g_hbm.at[:, pl.ds(0, CHS)], buf, sem).wait()

    def init_lists(i, _):
        idx = i * 16 + iota
        plsc.store_scatter(srco, [idx], zero16i)
        plsc.store_scatter(ldsto, [idx], zero16i + NB)
        return 0
    lax.fori_loop(0, CAP // 16, init_lists, 0)

    def init_hist(i, _):
        plsc.store_scatter(hist, [iota, zero16i + i], zero16f)
        return 0
    lax.fori_loop(0, NB, init_hist, 0)

    def scan(buf, cursor):
        def grp(j, cur):
            srcv = plsc.load_gather(buf, [zero16i, j * 16 + iota])
            dstv = plsc.load_gather(buf, [zero16i + 1, j * 16 + iota])
            ldst = dstv - lo
            m = (ldst >= 0) & (ldst < NB)
            mi = jnp.where(m, 1, 0).astype(jnp.int32)
            pos = plsc.cumsum(mi) - mi
            idx = cur + pos
            plsc.store_scatter(srco, [idx], srcv, mask=m)
            plsc.store_scatter(ldsto, [idx], ldst, mask=m)
            plsc.addupdate_scatter(hist, [iota, ldst], zero16f + 1.0, mask=m)
            return cur + jnp.sum(mi)
        return lax.fori_loop(0, CHS // 16, grp, cursor)

    start(0, gb0, sem0)

    def pair(p, cursor):
        start(2 * p + 1, gb1, sem1)
        wait(gb0, sem0)
        cursor = scan(gb0, cursor)
        start(2 * p + 2, gb0, sem0)
        wait(gb1, sem1)
        return scan(gb1, cursor)
    cursor = lax.fori_loop(0, NC_SETUP // 2 - 1, pair, jnp.int32(0))

    start(NC_SETUP - 1, gb1, sem1)
    wait(gb0, sem0)
    cursor = scan(gb0, cursor)
    wait(gb1, sem1)
    cursor = scan(gb1, cursor)

    cpad = ((cursor + (2 * CH - 1)) // (2 * CH)) * (2 * CH)
    cpad = jnp.maximum(cpad, 2 * CH)

    plsc.store_scatter(cntb, [iota], zero16i + cpad)

    def dred(rb, _):
        base = rb * 16 + iota
        acc = zero16f
        for j in range(16):
            acc = acc + plsc.load_gather(hist, [zero16i + j, base])
        plsc.store_scatter(degb, [base], acc)
        return 0
    lax.fori_loop(0, NB // 16, dred, 0)

    pltpu.sync_copy(srco, srcl_hbm.at[wid])
    pltpu.sync_copy(ldsto, ldstl_hbm.at[wid])
    pltpu.sync_copy(cntb, cnt_hbm.at[wid])
    pltpu.sync_copy(degb, deg_hbm.at[pl.ds(lo, NB)])


@functools.partial(
    pl.kernel,
    out_type=(
        jax.ShapeDtypeStruct((NPAD, D), jnp.float32),
        jax.ShapeDtypeStruct((NPAD, D), jnp.float32),
        jax.ShapeDtypeStruct((NPAD, D), jnp.float32),
    ),
    mesh=_mesh,
    compiler_params=_sc_params,
    scratch_types=[
        pltpu.VMEM((CAP,), jnp.int32),
        pltpu.VMEM((CAP,), jnp.int32),
        pltpu.VMEM((CH, D), jnp.float32),
        pltpu.VMEM((CH, D), jnp.float32),
    ] + [pltpu.VMEM((NBP, 16), jnp.float32) for _ in range(12)] + [
        pltpu.VMEM((16,), jnp.int32),
        pltpu.SemaphoreType.DMA,
        pltpu.SemaphoreType.DMA,
    ],
)
def _sc_agg(x_hbm, srcl_hbm, ldstl_hbm, cnt_hbm, sum_hbm, max_hbm, min_hbm,
            srcb, ldstb, rows0, rows1,
            s0, s1, s2, s3, m0, m1, m2, m3, n0, n1, n2, n3,
            cntb, sem0, sem1):
    acc_s = [s0, s1, s2, s3]
    acc_m = [m0, m1, m2, m3]
    acc_n = [n0, n1, n2, n3]
    cid = lax.axis_index("c")
    sid = lax.axis_index("s")
    wid = sid * 2 + cid
    iota = lax.iota(jnp.int32, 16)
    zero16i = jnp.zeros((16,), jnp.int32)
    zero16f = jnp.zeros((16,), jnp.float32)

    pltpu.sync_copy(srcl_hbm.at[wid], srcb)
    pltpu.sync_copy(ldstl_hbm.at[wid], ldstb)
    pltpu.sync_copy(cnt_hbm.at[wid], cntb)
    cnt = jnp.max(plsc.load_gather(cntb, [iota]))

    def start(ci, buf, sem):
        pltpu.async_copy(x_hbm.at[srcb.at[pl.ds(ci * CH, CH)]], buf, sem)

    def wait(buf, sem):
        pltpu.make_async_copy(x_hbm.at[srcb.at[pl.ds(0, CH)]], buf, sem).wait()

    def ibody(r, _):
        rr = zero16i + r
        for k in range(D // 16):
            plsc.store_scatter(acc_s[k], [rr, iota], zero16f)
            plsc.store_scatter(acc_m[k], [rr, iota], zero16f + _NEG)
            plsc.store_scatter(acc_n[k], [rr, iota], zero16f + _POS)
        return 0
    lax.fori_loop(0, NBP, ibody, 0)

    def proc(rows, ci):
        base = ci * CH

        def grp(j, _):
            ldv = plsc.load_gather(ldstb, [base + j * 16 + iota])
            for e in range(16):
                row = _lane_bcast(ldv, e)
                er = zero16i + (j * 16 + e)
                for k in range(D // 16):
                    col = k * 16 + iota
                    mv = plsc.load_gather(rows, [er, col])
                    plsc.addupdate_scatter(acc_s[k], [row, iota], mv)
                    cm = plsc.load_gather(acc_m[k], [row, iota])
                    plsc.store_scatter(acc_m[k], [row, iota],
                                       jnp.maximum(cm, mv))
                    cn = plsc.load_gather(acc_n[k], [row, iota])
                    plsc.store_scatter(acc_n[k], [row, iota],
                                       jnp.minimum(cn, mv))
            return 0
        lax.fori_loop(0, CH // 16, grp, 0)

    nc = cnt // CH
    start(0, rows0, sem0)

    def pair(p, _):
        start(2 * p + 1, rows1, sem1)
        wait(rows0, sem0)
        proc(rows0, 2 * p)
        start(2 * p + 2, rows0, sem0)
        wait(rows1, sem1)
        proc(rows1, 2 * p + 1)
        return 0
    lax.fori_loop(0, nc // 2 - 1, pair, 0)

    start(nc - 1, rows1, sem1)
    wait(rows0, sem0)
    proc(rows0, nc - 2)
    wait(rows1, sem1)
    proc(rows1, nc - 1)

    for k in range(D // 16):
        ks = pl.ds(k * 16, 16)
        pltpu.sync_copy(acc_s[k].at[pl.ds(0, NB)],
                        sum_hbm.at[pl.ds(wid * NB, NB), ks])
        pltpu.sync_copy(acc_m[k].at[pl.ds(0, NB)],
                        max_hbm.at[pl.ds(wid * NB, NB), ks])
        pltpu.sync_copy(acc_n[k].at[pl.ds(0, NB)],
                        min_hbm.at[pl.ds(wid * NB, NB), ks])


_BR = 2048


def _embed_body(h_ref, w_ref, b_ref, o_ref):
    o_ref[...] = jnp.dot(h_ref[...], w_ref[...],
                         preferred_element_type=jnp.float32) + b_ref[...]


def _embed(hpad, w, b):
    return pl.pallas_call(
        _embed_body,
        out_shape=jax.ShapeDtypeStruct((NPAD, D), jnp.float32),
        grid=(NPAD // _BR,),
        in_specs=[
            pl.BlockSpec((_BR, IN_DIM), lambda i: (i, 0)),
            pl.BlockSpec((IN_DIM, D), lambda i: (0, 0)),
            pl.BlockSpec((1, D), lambda i: (0, 0)),
        ],
        out_specs=pl.BlockSpec((_BR, D), lambda i: (i, 0)),
    )(hpad, w, b)


def _combine_body(x_ref, s_ref, mx_ref, mn_ref, deg_ref, sn_ref, w_ref, b_ref,
                  o_ref):
    deg = deg_ref[...]
    x = x_ref[...]
    pos = deg > 0.0
    mean = s_ref[...] / jnp.maximum(deg, 1.0)
    mx = jnp.where(pos, mx_ref[...], 0.0)
    mn = jnp.where(pos, mn_ref[...], 0.0)
    la = jnp.log(deg + 1.0)
    amp = la / AVG_LOG
    att = jnp.where(la > 0.0, AVG_LOG / jnp.maximum(la, 1e-6), 1.0)
    cat = jnp.concatenate(
        [x, mean, mean * amp, mean * att, mx, mx * amp, mx * att,
         mn, mn * amp, mn * att], axis=1)
    out = jnp.dot(cat, w_ref[...], preferred_element_type=jnp.float32)
    out = out + b_ref[...]
    out = jnp.maximum(out * sn_ref[...], 0.0) + x
    o_ref[...] = out


def _combine(x, s, mx, mn, deg, sn, w, b):
    return pl.pallas_call(
        _combine_body,
        out_shape=jax.ShapeDtypeStruct((NPAD, D), jnp.float32),
        grid=(NPAD // _BR,),
        in_specs=[
            pl.BlockSpec((_BR, D), lambda i: (i, 0)),
            pl.BlockSpec((_BR, D), lambda i: (i, 0)),
            pl.BlockSpec((_BR, D), lambda i: (i, 0)),
            pl.BlockSpec((_BR, D), lambda i: (i, 0)),
            pl.BlockSpec((_BR, 1), lambda i: (i, 0)),
            pl.BlockSpec((_BR, 1), lambda i: (i, 0)),
            pl.BlockSpec((CAT_DIM, D), lambda i: (0, 0)),
            pl.BlockSpec((1, D), lambda i: (0, 0)),
        ],
        out_specs=pl.BlockSpec((_BR, D), lambda i: (i, 0)),
    )(x, s, mx, mn, deg, sn, w, b)


def _readout_body(x_ref, w0_ref, b0_ref, w1_ref, b1_ref, w2_ref, b2_ref,
                  o_ref):
    y = jnp.dot(x_ref[...], w0_ref[...], preferred_element_type=jnp.float32)
    y = jnp.maximum(y + b0_ref[...], 0.0)
    y = jnp.dot(y, w1_ref[...], preferred_element_type=jnp.float32)
    y = jnp.maximum(y + b1_ref[...], 0.0)
    y = jnp.dot(y, w2_ref[...], preferred_element_type=jnp.float32)
    o_ref[...] = y + b2_ref[...]


def _readout(x, w0, b0, w1, b1, w2, b2):
    return pl.pallas_call(
        _readout_body,
        out_shape=jax.ShapeDtypeStruct((NPAD, 1), jnp.float32),
        grid=(NPAD // _BR,),
        in_specs=[
            pl.BlockSpec((_BR, D), lambda i: (i, 0)),
            pl.BlockSpec((D, 32), lambda i: (0, 0)),
            pl.BlockSpec((1, 32), lambda i: (0, 0)),
            pl.BlockSpec((32, 16), lambda i: (0, 0)),
            pl.BlockSpec((1, 16), lambda i: (0, 0)),
            pl.BlockSpec((16, 1), lambda i: (0, 0)),
            pl.BlockSpec((1, 1), lambda i: (0, 0)),
        ],
        out_specs=pl.BlockSpec((_BR, 1), lambda i: (i, 0)),
    )(x, w0, b0, w1, b1, w2, b2)


def kernel(g, h, e, snorm_n, snorm_e, W_emb, b_emb, W_l0, b_l0, W_l1, b_l1,
           W_l2, b_l2, W_l3, b_l3, W_r0, b_r0, W_r1, b_r1, W_r2, b_r2):
    srcl, ldstl, cnts, deg = _sc_setup(g)
    deg2 = deg[:, None]
    hpad = jnp.pad(h, ((0, NPAD - N), (0, 0)))
    snp = jnp.pad(snorm_n, ((0, NPAD - N), (0, 0)))
    x = _embed(hpad, W_emb, b_emb[None, :])
    for W, b in ((W_l0, b_l0), (W_l1, b_l1), (W_l2, b_l2), (W_l3, b_l3)):
        s, mx, mn = _sc_agg(x, srcl, ldstl, cnts)
        x = _combine(x, s, mx, mn, deg2, snp, W, b[None, :])
    y = _readout(x, W_r0, b_r0[None, :], W_r1, b_r1[None, :],
                 W_r2, b_r2[None, :])
    return y[:N]


# speedup vs baseline: 1.0781x; 1.0781x over previous
import functools

import jax
import jax.numpy as jnp
from jax import lax
from jax.experimental import pallas as pl
from jax.experimental.pallas import tpu as pltpu
from jax.experimental.pallas import tpu_sc as plsc

N = 10000
E = 320000
IN_DIM = 128
D = 64
AVG_LOG = 3.4965075614664802
CAT_DIM = 10 * D

NT = 32
NB = 320
NPAD = NT * NB
NBP = NB + 8
CAP = 12288
CH = 128
CHS = 2000

_NEG = -3.0e38
_POS = 3.0e38

_mesh = plsc.VectorSubcoreMesh(core_axis_name="c", subcore_axis_name="s")
_sc_params = pltpu.CompilerParams(needs_layout_passes=False,
                                  use_tc_tiling_on_sc=False)


def _lane_bcast(v, j):
    idx = jnp.full((16, 1), j, jnp.int32)
    return lax.gather(
        v, idx,
        lax.GatherDimensionNumbers(
            offset_dims=(), collapsed_slice_dims=(0,), start_index_map=(0,)),
        (1,), mode=lax.GatherScatterMode.PROMISE_IN_BOUNDS)


NC_SETUP = E // CHS


@functools.partial(
    pl.kernel,
    out_type=(
        jax.ShapeDtypeStruct((NT, CAP), jnp.int32),
        jax.ShapeDtypeStruct((NT, CAP), jnp.int32),
        jax.ShapeDtypeStruct((NT, 16), jnp.int32),
        jax.ShapeDtypeStruct((NPAD,), jnp.float32),
    ),
    mesh=_mesh,
    compiler_params=_sc_params,
    scratch_types=[
        pltpu.VMEM((2, CHS), jnp.int32),
        pltpu.VMEM((2, CHS), jnp.int32),
        pltpu.VMEM((CAP,), jnp.int32),
        pltpu.VMEM((CAP,), jnp.int32),
        pltpu.VMEM((16, NB), jnp.float32),
        pltpu.VMEM((NB,), jnp.float32),
        pltpu.VMEM((16,), jnp.int32),
        pltpu.SemaphoreType.DMA,
        pltpu.SemaphoreType.DMA,
    ],
)
def _sc_setup(g_hbm, srcl_hbm, ldstl_hbm, cnt_hbm, deg_hbm,
              gb0, gb1, srco, ldsto, hist, degb, cntb, sem0, sem1):
    cid = lax.axis_index("c")
    sid = lax.axis_index("s")
    wid = sid * 2 + cid
    lo = wid * NB
    iota = lax.iota(jnp.int32, 16)
    zero16f = jnp.zeros((16,), jnp.float32)
    zero16i = jnp.zeros((16,), jnp.int32)

    def start(ci, buf, sem):
        pltpu.async_copy(g_hbm.at[:, pl.ds(ci * CHS, CHS)], buf, sem)

    def wait(buf, sem):
        pltpu.make_async_copy(g_hbm.at[:, pl.ds(0, CHS)], buf, sem).wait()

    def init_lists(i, _):
        idx = i * 16 + iota
        plsc.store_scatter(srco, [idx], zero16i)
        plsc.store_scatter(ldsto, [idx], zero16i + NB)
        return 0
    lax.fori_loop(0, CAP // 16, init_lists, 0)

    def init_hist(i, _):
        plsc.store_scatter(hist, [iota, zero16i + i], zero16f)
        return 0
    lax.fori_loop(0, NB, init_hist, 0)

    def scan(buf, cursor):
        def grp(j, cur):
            srcv = plsc.load_gather(buf, [zero16i, j * 16 + iota])
            dstv = plsc.load_gather(buf, [zero16i + 1, j * 16 + iota])
            ldst = dstv - lo
            m = (ldst >= 0) & (ldst < NB)
            mi = jnp.where(m, 1, 0).astype(jnp.int32)
            pos = plsc.cumsum(mi) - mi
            idx = cur + pos
            plsc.store_scatter(srco, [idx], srcv, mask=m)
            plsc.store_scatter(ldsto, [idx], ldst, mask=m)
            plsc.addupdate_scatter(hist, [iota, ldst], zero16f + 1.0, mask=m)
            return cur + jnp.sum(mi)
        return plsc.parallel_loop(0, CHS // 16, carry=cursor)(grp)

    start(0, gb0, sem0)

    def pair(p, cursor):
        start(2 * p + 1, gb1, sem1)
        wait(gb0, sem0)
        cursor = scan(gb0, cursor)
        start(2 * p + 2, gb0, sem0)
        wait(gb1, sem1)
        return scan(gb1, cursor)
    cursor = lax.fori_loop(0, NC_SETUP // 2 - 1, pair, jnp.int32(0))

    start(NC_SETUP - 1, gb1, sem1)
    wait(gb0, sem0)
    cursor = scan(gb0, cursor)
    wait(gb1, sem1)
    cursor = scan(gb1, cursor)

    cpad = ((cursor + (2 * CH - 1)) // (2 * CH)) * (2 * CH)
    cpad = jnp.maximum(cpad, 2 * CH)

    plsc.store_scatter(cntb, [iota], zero16i + cpad)

    def dred(rb, _):
        base = rb * 16 + iota
        acc = zero16f
        for j in range(16):
            acc = acc + plsc.load_gather(hist, [zero16i + j, base])
        plsc.store_scatter(degb, [base], acc)
        return 0
    lax.fori_loop(0, NB // 16, dred, 0)

    pltpu.sync_copy(srco, srcl_hbm.at[wid])
    pltpu.sync_copy(ldsto, ldstl_hbm.at[wid])
    pltpu.sync_copy(cntb, cnt_hbm.at[wid])
    pltpu.sync_copy(degb, deg_hbm.at[pl.ds(lo, NB)])


@functools.partial(
    pl.kernel,
    out_type=(
        jax.ShapeDtypeStruct((NPAD, D), jnp.float32),
        jax.ShapeDtypeStruct((NPAD, D), jnp.float32),
        jax.ShapeDtypeStruct((NPAD, D), jnp.float32),
    ),
    mesh=_mesh,
    compiler_params=_sc_params,
    scratch_types=[
        pltpu.VMEM((CAP,), jnp.int32),
        pltpu.VMEM((CAP,), jnp.int32),
        pltpu.VMEM((CH, D), jnp.float32),
        pltpu.VMEM((CH, D), jnp.float32),
    ] + [pltpu.VMEM((NBP, 16), jnp.float32) for _ in range(12)] + [
        pltpu.VMEM((16,), jnp.int32),
        pltpu.SemaphoreType.DMA,
        pltpu.SemaphoreType.DMA,
    ],
)
def _sc_agg(x_hbm, srcl_hbm, ldstl_hbm, cnt_hbm, sum_hbm, max_hbm, min_hbm,
            srcb, ldstb, rows0, rows1,
            s0, s1, s2, s3, m0, m1, m2, m3, n0, n1, n2, n3,
            cntb, sem0, sem1):
    acc_s = [s0, s1, s2, s3]
    acc_m = [m0, m1, m2, m3]
    acc_n = [n0, n1, n2, n3]
    cid = lax.axis_index("c")
    sid = lax.axis_index("s")
    wid = sid * 2 + cid
    iota = lax.iota(jnp.int32, 16)
    zero16i = jnp.zeros((16,), jnp.int32)
    zero16f = jnp.zeros((16,), jnp.float32)

    pltpu.sync_copy(srcl_hbm.at[wid], srcb)
    pltpu.sync_copy(ldstl_hbm.at[wid], ldstb)
    pltpu.sync_copy(cnt_hbm.at[wid], cntb)
    cnt = jnp.max(plsc.load_gather(cntb, [iota]))

    def start(ci, buf, sem):
        pltpu.async_copy(x_hbm.at[srcb.at[pl.ds(ci * CH, CH)]], buf, sem)

    def wait(buf, sem):
        pltpu.make_async_copy(x_hbm.at[srcb.at[pl.ds(0, CH)]], buf, sem).wait()

    def ibody(r, _):
        rr = zero16i + r
        for k in range(D // 16):
            plsc.store_scatter(acc_s[k], [rr, iota], zero16f)
            plsc.store_scatter(acc_m[k], [rr, iota], zero16f + _NEG)
            plsc.store_scatter(acc_n[k], [rr, iota], zero16f + _POS)
        return 0
    lax.fori_loop(0, NBP, ibody, 0)

    def proc(rows, ci):
        base = ci * CH

        def grp(j, _):
            ldv = plsc.load_gather(ldstb, [base + j * 16 + iota])
            for e in range(16):
                row = _lane_bcast(ldv, e)
                er = zero16i + (j * 16 + e)
                for k in range(D // 16):
                    col = k * 16 + iota
                    mv = plsc.load_gather(rows, [er, col])
                    plsc.addupdate_scatter(acc_s[k], [row, iota], mv)
                    cm = plsc.load_gather(acc_m[k], [row, iota])
                    plsc.store_scatter(acc_m[k], [row, iota],
                                       jnp.maximum(cm, mv))
                    cn = plsc.load_gather(acc_n[k], [row, iota])
                    plsc.store_scatter(acc_n[k], [row, iota],
                                       jnp.minimum(cn, mv))
            return 0
        lax.fori_loop(0, CH // 16, grp, 0)

    nc = cnt // CH
    start(0, rows0, sem0)

    def pair(p, _):
        start(2 * p + 1, rows1, sem1)
        wait(rows0, sem0)
        proc(rows0, 2 * p)
        start(2 * p + 2, rows0, sem0)
        wait(rows1, sem1)
        proc(rows1, 2 * p + 1)
        return 0
    lax.fori_loop(0, nc // 2 - 1, pair, 0)

    start(nc - 1, rows1, sem1)
    wait(rows0, sem0)
    proc(rows0, nc - 2)
    wait(rows1, sem1)
    proc(rows1, nc - 1)

    for k in range(D // 16):
        ks = pl.ds(k * 16, 16)
        pltpu.sync_copy(acc_s[k].at[pl.ds(0, NB)],
                        sum_hbm.at[pl.ds(wid * NB, NB), ks])
        pltpu.sync_copy(acc_m[k].at[pl.ds(0, NB)],
                        max_hbm.at[pl.ds(wid * NB, NB), ks])
        pltpu.sync_copy(acc_n[k].at[pl.ds(0, NB)],
                        min_hbm.at[pl.ds(wid * NB, NB), ks])


_BR = 2048


def _embed_body(h_ref, w_ref, b_ref, o_ref):
    o_ref[...] = jnp.dot(h_ref[...], w_ref[...],
                         preferred_element_type=jnp.float32) + b_ref[...]


def _embed(hpad, w, b):
    return pl.pallas_call(
        _embed_body,
        out_shape=jax.ShapeDtypeStruct((NPAD, D), jnp.float32),
        grid=(NPAD // _BR,),
        in_specs=[
            pl.BlockSpec((_BR, IN_DIM), lambda i: (i, 0)),
            pl.BlockSpec((IN_DIM, D), lambda i: (0, 0)),
            pl.BlockSpec((1, D), lambda i: (0, 0)),
        ],
        out_specs=pl.BlockSpec((_BR, D), lambda i: (i, 0)),
    )(hpad, w, b)


def _combine_body(x_ref, s_ref, mx_ref, mn_ref, deg_ref, sn_ref, w_ref, b_ref,
                  o_ref):
    deg = deg_ref[...]
    x = x_ref[...]
    pos = deg > 0.0
    mean = s_ref[...] / jnp.maximum(deg, 1.0)
    mx = jnp.where(pos, mx_ref[...], 0.0)
    mn = jnp.where(pos, mn_ref[...], 0.0)
    la = jnp.log(deg + 1.0)
    amp = la / AVG_LOG
    att = jnp.where(la > 0.0, AVG_LOG / jnp.maximum(la, 1e-6), 1.0)
    cat = jnp.concatenate(
        [x, mean, mean * amp, mean * att, mx, mx * amp, mx * att,
         mn, mn * amp, mn * att], axis=1)
    out = jnp.dot(cat, w_ref[...], preferred_element_type=jnp.float32)
    out = out + b_ref[...]
    out = jnp.maximum(out * sn_ref[...], 0.0) + x
    o_ref[...] = out


def _combine(x, s, mx, mn, deg, sn, w, b):
    return pl.pallas_call(
        _combine_body,
        out_shape=jax.ShapeDtypeStruct((NPAD, D), jnp.float32),
        grid=(NPAD // _BR,),
        in_specs=[
            pl.BlockSpec((_BR, D), lambda i: (i, 0)),
            pl.BlockSpec((_BR, D), lambda i: (i, 0)),
            pl.BlockSpec((_BR, D), lambda i: (i, 0)),
            pl.BlockSpec((_BR, D), lambda i: (i, 0)),
            pl.BlockSpec((_BR, 1), lambda i: (i, 0)),
            pl.BlockSpec((_BR, 1), lambda i: (i, 0)),
            pl.BlockSpec((CAT_DIM, D), lambda i: (0, 0)),
            pl.BlockSpec((1, D), lambda i: (0, 0)),
        ],
        out_specs=pl.BlockSpec((_BR, D), lambda i: (i, 0)),
    )(x, s, mx, mn, deg, sn, w, b)


def _readout_body(x_ref, w0_ref, b0_ref, w1_ref, b1_ref, w2_ref, b2_ref,
                  o_ref):
    y = jnp.dot(x_ref[...], w0_ref[...], preferred_element_type=jnp.float32)
    y = jnp.maximum(y + b0_ref[...], 0.0)
    y = jnp.dot(y, w1_ref[...], preferred_element_type=jnp.float32)
    y = jnp.maximum(y + b1_ref[...], 0.0)
    y = jnp.dot(y, w2_ref[...], preferred_element_type=jnp.float32)
    o_ref[...] = y + b2_ref[...]


def _readout(x, w0, b0, w1, b1, w2, b2):
    return pl.pallas_call(
        _readout_body,
        out_shape=jax.ShapeDtypeStruct((NPAD, 1), jnp.float32),
        grid=(NPAD // _BR,),
        in_specs=[
            pl.BlockSpec((_BR, D), lambda i: (i, 0)),
            pl.BlockSpec((D, 32), lambda i: (0, 0)),
            pl.BlockSpec((1, 32), lambda i: (0, 0)),
            pl.BlockSpec((32, 16), lambda i: (0, 0)),
            pl.BlockSpec((1, 16), lambda i: (0, 0)),
            pl.BlockSpec((16, 1), lambda i: (0, 0)),
            pl.BlockSpec((1, 1), lambda i: (0, 0)),
        ],
        out_specs=pl.BlockSpec((_BR, 1), lambda i: (i, 0)),
    )(x, w0, b0, w1, b1, w2, b2)


def kernel(g, h, e, snorm_n, snorm_e, W_emb, b_emb, W_l0, b_l0, W_l1, b_l1,
           W_l2, b_l2, W_l3, b_l3, W_r0, b_r0, W_r1, b_r1, W_r2, b_r2):
    srcl, ldstl, cnts, deg = _sc_setup(g)
    deg2 = deg[:, None]
    hpad = jnp.pad(h, ((0, NPAD - N), (0, 0)))
    snp = jnp.pad(snorm_n, ((0, NPAD - N), (0, 0)))
    x = _embed(hpad, W_emb, b_emb[None, :])
    for W, b in ((W_l0, b_l0), (W_l1, b_l1), (W_l2, b_l2), (W_l3, b_l3)):
        s, mx, mn = _sc_agg(x, srcl, ldstl, cnts)
        x = _combine(x, s, mx, mn, deg2, snp, W, b[None, :])
    y = _readout(x, W_r0, b_r0[None, :], W_r1, b_r1[None, :],
                 W_r2, b_r2[None, :])
    return y[:N]


# speedup vs baseline: 2.2393x; 2.0771x over previous
import functools

import jax
import jax.numpy as jnp
from jax import lax
from jax.experimental import pallas as pl
from jax.experimental.pallas import tpu as pltpu
from jax.experimental.pallas import tpu_sc as plsc

N = 10000
E = 320000
IN_DIM = 128
D = 64
AVG_LOG = 3.4965075614664802
CAT_DIM = 10 * D

NT = 32
NB = 320
NPAD = NT * NB
NBP = NB + 8
CAP = 12288
CH = 128
CHS = 2000

_NEG = -3.0e38
_POS = 3.0e38
_CNT_BASE = 1

_mesh = plsc.VectorSubcoreMesh(core_axis_name="c", subcore_axis_name="s")
_sc_params = pltpu.CompilerParams(needs_layout_passes=False,
                                  use_tc_tiling_on_sc=False)


def _lane_bcast(v, j):
    idx = jnp.full((16, 1), j, jnp.int32)
    return lax.gather(
        v, idx,
        lax.GatherDimensionNumbers(
            offset_dims=(), collapsed_slice_dims=(0,), start_index_map=(0,)),
        (1,), mode=lax.GatherScatterMode.PROMISE_IN_BOUNDS)


NC_SETUP = E // CHS


@functools.partial(
    pl.kernel,
    out_type=(
        jax.ShapeDtypeStruct((NT, CAP), jnp.int32),
        jax.ShapeDtypeStruct((NT, CAP), jnp.int32),
        jax.ShapeDtypeStruct((NT, 16), jnp.int32),
        jax.ShapeDtypeStruct((NPAD,), jnp.float32),
    ),
    mesh=_mesh,
    compiler_params=_sc_params,
    scratch_types=[
        pltpu.VMEM((2, CHS), jnp.int32),
        pltpu.VMEM((2, CHS), jnp.int32),
        pltpu.VMEM((CAP,), jnp.int32),
        pltpu.VMEM((CAP,), jnp.int32),
        pltpu.VMEM((16, NB), jnp.float32),
        pltpu.VMEM((NB,), jnp.float32),
        pltpu.VMEM((16,), jnp.int32),
        pltpu.VMEM((NB + 16,), jnp.int32),
        pltpu.VMEM((CAP,), jnp.int32),
        pltpu.VMEM((CAP,), jnp.int32),
        pltpu.SemaphoreType.DMA,
        pltpu.SemaphoreType.DMA,
    ],
)
def _sc_setup(g_hbm, srcl_hbm, ldstl_hbm, cnt_hbm, deg_hbm,
              gb0, gb1, srco, ldsto, hist, degb, cntb, offs, srcs, ldsts,
              sem0, sem1):
    cid = lax.axis_index("c")
    sid = lax.axis_index("s")
    wid = sid * 2 + cid
    lo = wid * NB
    iota = lax.iota(jnp.int32, 16)
    zero16f = jnp.zeros((16,), jnp.float32)
    zero16i = jnp.zeros((16,), jnp.int32)

    def start(ci, buf, sem):
        pltpu.async_copy(g_hbm.at[:, pl.ds(ci * CHS, CHS)], buf, sem)

    def wait(buf, sem):
        pltpu.make_async_copy(g_hbm.at[:, pl.ds(0, CHS)], buf, sem).wait()

    def init_lists(i, _):
        idx = i * 16 + iota
        plsc.store_scatter(srco, [idx], zero16i)
        plsc.store_scatter(ldsto, [idx], zero16i + NB)
        return 0
    lax.fori_loop(0, CAP // 16, init_lists, 0)

    def init_hist(i, _):
        plsc.store_scatter(hist, [iota, zero16i + i], zero16f)
        return 0
    lax.fori_loop(0, NB, init_hist, 0)

    def scan(buf, cursor):
        def grp(j, cur):
            srcv = plsc.load_gather(buf, [zero16i, j * 16 + iota])
            dstv = plsc.load_gather(buf, [zero16i + 1, j * 16 + iota])
            ldst = dstv - lo
            m = (ldst >= 0) & (ldst < NB)
            mi = jnp.where(m, 1, 0).astype(jnp.int32)
            pos = plsc.cumsum(mi) - mi
            idx = cur + pos
            plsc.store_scatter(srco, [idx], srcv, mask=m)
            plsc.store_scatter(ldsto, [idx], ldst, mask=m)
            plsc.addupdate_scatter(hist, [iota, ldst], zero16f + 1.0, mask=m)
            return cur + jnp.sum(mi)
        return plsc.parallel_loop(0, CHS // 16, carry=cursor)(grp)

    start(0, gb0, sem0)

    def pair(p, cursor):
        start(2 * p + 1, gb1, sem1)
        wait(gb0, sem0)
        cursor = scan(gb0, cursor)
        start(2 * p + 2, gb0, sem0)
        wait(gb1, sem1)
        return scan(gb1, cursor)
    cursor = lax.fori_loop(0, NC_SETUP // 2 - 1, pair, jnp.int32(0))

    start(NC_SETUP - 1, gb1, sem1)
    wait(gb0, sem0)
    cursor = scan(gb0, cursor)
    wait(gb1, sem1)
    cursor = scan(gb1, cursor)

    cpad = ((cursor + (2 * CH - 1)) // (2 * CH)) * (2 * CH)
    cpad = jnp.maximum(cpad, 2 * CH)

    plsc.store_scatter(cntb, [iota], zero16i + cpad)

    def dred(rb, run):
        base = rb * 16 + iota
        acc = zero16f
        for j in range(16):
            acc = acc + plsc.load_gather(hist, [zero16i + j, base])
        plsc.store_scatter(degb, [base], acc)
        di = acc.astype(jnp.int32)
        cs = plsc.cumsum(di)
        plsc.store_scatter(offs, [base], run + cs - di)
        return run + jnp.max(cs)
    lax.fori_loop(0, NB // 16, dred, jnp.int32(0))
    plsc.store_scatter(offs, [NB + iota], zero16i + cursor)

    def sortg(j, _):
        idx = j * 16 + iota
        ldv = plsc.load_gather(ldsto, [idx])
        sv = plsc.load_gather(srco, [idx])
        cntv, lastm = plsc.scan_count(ldv)
        basev = plsc.load_gather(offs, [ldv])
        pos = basev + cntv - _CNT_BASE
        plsc.store_scatter(srcs, [pos], sv)
        plsc.store_scatter(ldsts, [pos], ldv)
        plsc.addupdate_scatter(offs, [ldv], cntv + (1 - _CNT_BASE),
                               mask=lastm)
        return 0
    lax.fori_loop(0, cpad // 16, sortg, 0)

    pltpu.sync_copy(srcs, srcl_hbm.at[wid])
    pltpu.sync_copy(ldsts, ldstl_hbm.at[wid])
    pltpu.sync_copy(cntb, cnt_hbm.at[wid])
    pltpu.sync_copy(degb, deg_hbm.at[pl.ds(lo, NB)])


@functools.partial(
    pl.kernel,
    out_type=(
        jax.ShapeDtypeStruct((NPAD, D), jnp.float32),
        jax.ShapeDtypeStruct((NPAD, D), jnp.float32),
        jax.ShapeDtypeStruct((NPAD, D), jnp.float32),
    ),
    mesh=_mesh,
    compiler_params=_sc_params,
    scratch_types=[
        pltpu.VMEM((CAP,), jnp.int32),
        pltpu.VMEM((CAP,), jnp.int32),
        pltpu.VMEM((CH, D), jnp.float32),
        pltpu.VMEM((CH, D), jnp.float32),
    ] + [pltpu.VMEM((NBP, 16), jnp.float32) for _ in range(12)] + [
        pltpu.VMEM((16,), jnp.int32),
        pltpu.SemaphoreType.DMA,
        pltpu.SemaphoreType.DMA,
    ],
)
def _sc_agg(x_hbm, srcl_hbm, ldstl_hbm, cnt_hbm, sum_hbm, max_hbm, min_hbm,
            srcb, ldstb, rows0, rows1,
            s0, s1, s2, s3, m0, m1, m2, m3, n0, n1, n2, n3,
            cntb, sem0, sem1):
    acc_s = [s0, s1, s2, s3]
    acc_m = [m0, m1, m2, m3]
    acc_n = [n0, n1, n2, n3]
    cid = lax.axis_index("c")
    sid = lax.axis_index("s")
    wid = sid * 2 + cid
    iota = lax.iota(jnp.int32, 16)
    zero16i = jnp.zeros((16,), jnp.int32)
    zero16f = jnp.zeros((16,), jnp.float32)

    pltpu.sync_copy(srcl_hbm.at[wid], srcb)
    pltpu.sync_copy(ldstl_hbm.at[wid], ldstb)
    pltpu.sync_copy(cnt_hbm.at[wid], cntb)
    cnt = jnp.max(plsc.load_gather(cntb, [iota]))

    def start(ci, buf, sem):
        pltpu.async_copy(x_hbm.at[srcb.at[pl.ds(ci * CH, CH)]], buf, sem)

    def wait(buf, sem):
        pltpu.make_async_copy(x_hbm.at[srcb.at[pl.ds(0, CH)]], buf, sem).wait()

    def ibody(r, _):
        rr = zero16i + r
        for k in range(D // 16):
            plsc.store_scatter(acc_s[k], [rr, iota], zero16f)
            plsc.store_scatter(acc_m[k], [rr, iota], zero16f + _NEG)
            plsc.store_scatter(acc_n[k], [rr, iota], zero16f + _POS)
        return 0
    lax.fori_loop(0, NBP, ibody, 0)

    neutral = ((zero16f,) * 4, (zero16f + _NEG,) * 4, (zero16f + _POS,) * 4)

    def flush(rs, rm, rn, curs):
        curv = zero16i + curs
        for k in range(D // 16):
            plsc.store_scatter(acc_s[k], [curv, iota], rs[k])
            plsc.store_scatter(acc_m[k], [curv, iota], rm[k])
            plsc.store_scatter(acc_n[k], [curv, iota], rn[k])

    def proc(rows, ci, state):
        base = ci * CH

        def grp(j, st):
            rs, rm, rn, curs = st
            ldv = plsc.load_gather(ldstb, [base + j * 16 + iota])
            for e in range(16):
                dv = _lane_bcast(ldv, e)
                dsc = jnp.max(dv)

                def do_flush(rs, rm, rn, curs):
                    flush(rs, rm, rn, curs)
                    return neutral + (dsc,)

                def no_flush(rs, rm, rn, curs):
                    return (rs, rm, rn, curs)

                rs, rm, rn, curs = lax.cond(dsc != curs, do_flush, no_flush,
                                            rs, rm, rn, curs)
                er = zero16i + (j * 16 + e)
                mvs = [plsc.load_gather(rows, [er, k * 16 + iota])
                       for k in range(D // 16)]
                rs = tuple(rs[k] + mvs[k] for k in range(D // 16))
                rm = tuple(jnp.maximum(rm[k], mvs[k])
                           for k in range(D // 16))
                rn = tuple(jnp.minimum(rn[k], mvs[k])
                           for k in range(D // 16))
            return (rs, rm, rn, curs)
        return lax.fori_loop(0, CH // 16, grp, state)

    nc = cnt // CH
    state = neutral + (jnp.int32(NB),)
    start(0, rows0, sem0)

    def pair(p, st):
        start(2 * p + 1, rows1, sem1)
        wait(rows0, sem0)
        st = proc(rows0, 2 * p, st)
        start(2 * p + 2, rows0, sem0)
        wait(rows1, sem1)
        return proc(rows1, 2 * p + 1, st)
    state = lax.fori_loop(0, nc // 2 - 1, pair, state)

    start(nc - 1, rows1, sem1)
    wait(rows0, sem0)
    state = proc(rows0, nc - 2, state)
    wait(rows1, sem1)
    state = proc(rows1, nc - 1, state)
    flush(*state)

    for k in range(D // 16):
        ks = pl.ds(k * 16, 16)
        pltpu.sync_copy(acc_s[k].at[pl.ds(0, NB)],
                        sum_hbm.at[pl.ds(wid * NB, NB), ks])
        pltpu.sync_copy(acc_m[k].at[pl.ds(0, NB)],
                        max_hbm.at[pl.ds(wid * NB, NB), ks])
        pltpu.sync_copy(acc_n[k].at[pl.ds(0, NB)],
                        min_hbm.at[pl.ds(wid * NB, NB), ks])


_BR = 2048


def _embed_body(h_ref, w_ref, b_ref, o_ref):
    o_ref[...] = jnp.dot(h_ref[...], w_ref[...],
                         preferred_element_type=jnp.float32) + b_ref[...]


def _embed(hpad, w, b):
    return pl.pallas_call(
        _embed_body,
        out_shape=jax.ShapeDtypeStruct((NPAD, D), jnp.float32),
        grid=(NPAD // _BR,),
        in_specs=[
            pl.BlockSpec((_BR, IN_DIM), lambda i: (i, 0)),
            pl.BlockSpec((IN_DIM, D), lambda i: (0, 0)),
            pl.BlockSpec((1, D), lambda i: (0, 0)),
        ],
        out_specs=pl.BlockSpec((_BR, D), lambda i: (i, 0)),
    )(hpad, w, b)


def _combine_body(x_ref, s_ref, mx_ref, mn_ref, deg_ref, sn_ref, w_ref, b_ref,
                  o_ref):
    deg = deg_ref[...]
    x = x_ref[...]
    pos = deg > 0.0
    mean = s_ref[...] / jnp.maximum(deg, 1.0)
    mx = jnp.where(pos, mx_ref[...], 0.0)
    mn = jnp.where(pos, mn_ref[...], 0.0)
    la = jnp.log(deg + 1.0)
    amp = la / AVG_LOG
    att = jnp.where(la > 0.0, AVG_LOG / jnp.maximum(la, 1e-6), 1.0)
    cat = jnp.concatenate(
        [x, mean, mean * amp, mean * att, mx, mx * amp, mx * att,
         mn, mn * amp, mn * att], axis=1)
    out = jnp.dot(cat, w_ref[...], preferred_element_type=jnp.float32)
    out = out + b_ref[...]
    out = jnp.maximum(out * sn_ref[...], 0.0) + x
    o_ref[...] = out


def _combine(x, s, mx, mn, deg, sn, w, b):
    return pl.pallas_call(
        _combine_body,
        out_shape=jax.ShapeDtypeStruct((NPAD, D), jnp.float32),
        grid=(NPAD // _BR,),
        in_specs=[
            pl.BlockSpec((_BR, D), lambda i: (i, 0)),
            pl.BlockSpec((_BR, D), lambda i: (i, 0)),
            pl.BlockSpec((_BR, D), lambda i: (i, 0)),
            pl.BlockSpec((_BR, D), lambda i: (i, 0)),
            pl.BlockSpec((_BR, 1), lambda i: (i, 0)),
            pl.BlockSpec((_BR, 1), lambda i: (i, 0)),
            pl.BlockSpec((CAT_DIM, D), lambda i: (0, 0)),
            pl.BlockSpec((1, D), lambda i: (0, 0)),
        ],
        out_specs=pl.BlockSpec((_BR, D), lambda i: (i, 0)),
    )(x, s, mx, mn, deg, sn, w, b)


def _readout_body(x_ref, w0_ref, b0_ref, w1_ref, b1_ref, w2_ref, b2_ref,
                  o_ref):
    y = jnp.dot(x_ref[...], w0_ref[...], preferred_element_type=jnp.float32)
    y = jnp.maximum(y + b0_ref[...], 0.0)
    y = jnp.dot(y, w1_ref[...], preferred_element_type=jnp.float32)
    y = jnp.maximum(y + b1_ref[...], 0.0)
    y = jnp.dot(y, w2_ref[...], preferred_element_type=jnp.float32)
    o_ref[...] = y + b2_ref[...]


def _readout(x, w0, b0, w1, b1, w2, b2):
    return pl.pallas_call(
        _readout_body,
        out_shape=jax.ShapeDtypeStruct((NPAD, 1), jnp.float32),
        grid=(NPAD // _BR,),
        in_specs=[
            pl.BlockSpec((_BR, D), lambda i: (i, 0)),
            pl.BlockSpec((D, 32), lambda i: (0, 0)),
            pl.BlockSpec((1, 32), lambda i: (0, 0)),
            pl.BlockSpec((32, 16), lambda i: (0, 0)),
            pl.BlockSpec((1, 16), lambda i: (0, 0)),
            pl.BlockSpec((16, 1), lambda i: (0, 0)),
            pl.BlockSpec((1, 1), lambda i: (0, 0)),
        ],
        out_specs=pl.BlockSpec((_BR, 1), lambda i: (i, 0)),
    )(x, w0, b0, w1, b1, w2, b2)


def kernel(g, h, e, snorm_n, snorm_e, W_emb, b_emb, W_l0, b_l0, W_l1, b_l1,
           W_l2, b_l2, W_l3, b_l3, W_r0, b_r0, W_r1, b_r1, W_r2, b_r2):
    srcl, ldstl, cnts, deg = _sc_setup(g)
    deg2 = deg[:, None]
    hpad = jnp.pad(h, ((0, NPAD - N), (0, 0)))
    snp = jnp.pad(snorm_n, ((0, NPAD - N), (0, 0)))
    x = _embed(hpad, W_emb, b_emb[None, :])
    for W, b in ((W_l0, b_l0), (W_l1, b_l1), (W_l2, b_l2), (W_l3, b_l3)):
        s, mx, mn = _sc_agg(x, srcl, ldstl, cnts)
        x = _combine(x, s, mx, mn, deg2, snp, W, b[None, :])
    y = _readout(x, W_r0, b_r0[None, :], W_r1, b_r1[None, :],
                 W_r2, b_r2[None, :])
    return y[:N]


# speedup vs baseline: 2.2856x; 1.0207x over previous
import functools

import jax
import jax.numpy as jnp
from jax import lax
from jax.experimental import pallas as pl
from jax.experimental.pallas import tpu as pltpu
from jax.experimental.pallas import tpu_sc as plsc

N = 10000
E = 320000
IN_DIM = 128
D = 64
AVG_LOG = 3.4965075614664802
CAT_DIM = 10 * D

NT = 32
NB = 320
NPAD = NT * NB
NBP = NB + 8
CAP = 12288
CH = 128
CHS = 2000

_NEG = -3.0e38
_POS = 3.0e38
_CNT_BASE = 1

_mesh = plsc.VectorSubcoreMesh(core_axis_name="c", subcore_axis_name="s")
_sc_params = pltpu.CompilerParams(needs_layout_passes=False,
                                  use_tc_tiling_on_sc=False)


def _lane_bcast(v, j):
    idx = jnp.full((16, 1), j, jnp.int32)
    return lax.gather(
        v, idx,
        lax.GatherDimensionNumbers(
            offset_dims=(), collapsed_slice_dims=(0,), start_index_map=(0,)),
        (1,), mode=lax.GatherScatterMode.PROMISE_IN_BOUNDS)


NC_SETUP = E // CHS


@functools.partial(
    pl.kernel,
    out_type=(
        jax.ShapeDtypeStruct((NT, CAP), jnp.int32),
        jax.ShapeDtypeStruct((NT, CAP), jnp.int32),
        jax.ShapeDtypeStruct((NT, 16), jnp.int32),
        jax.ShapeDtypeStruct((NPAD,), jnp.float32),
    ),
    mesh=_mesh,
    compiler_params=_sc_params,
    scratch_types=[
        pltpu.VMEM((2, CHS), jnp.int32),
        pltpu.VMEM((2, CHS), jnp.int32),
        pltpu.VMEM((CAP,), jnp.int32),
        pltpu.VMEM((CAP,), jnp.int32),
        pltpu.VMEM((16, NB), jnp.float32),
        pltpu.VMEM((NB,), jnp.float32),
        pltpu.VMEM((16,), jnp.int32),
        pltpu.VMEM((NB + 16,), jnp.int32),
        pltpu.VMEM((CAP,), jnp.int32),
        pltpu.VMEM((CAP,), jnp.int32),
        pltpu.SemaphoreType.DMA,
        pltpu.SemaphoreType.DMA,
    ],
)
def _sc_setup(g_hbm, srcl_hbm, ldstl_hbm, cnt_hbm, deg_hbm,
              gb0, gb1, srco, ldsto, hist, degb, cntb, offs, srcs, ldsts,
              sem0, sem1):
    cid = lax.axis_index("c")
    sid = lax.axis_index("s")
    wid = sid * 2 + cid
    lo = wid * NB
    iota = lax.iota(jnp.int32, 16)
    zero16f = jnp.zeros((16,), jnp.float32)
    zero16i = jnp.zeros((16,), jnp.int32)

    def start(ci, buf, sem):
        pltpu.async_copy(g_hbm.at[:, pl.ds(ci * CHS, CHS)], buf, sem)

    def wait(buf, sem):
        pltpu.make_async_copy(g_hbm.at[:, pl.ds(0, CHS)], buf, sem).wait()

    def init_lists(i, _):
        idx = i * 16 + iota
        plsc.store_scatter(srco, [idx], zero16i)
        plsc.store_scatter(ldsto, [idx], zero16i + NB)
        return 0
    lax.fori_loop(0, CAP // 16, init_lists, 0)

    def init_hist(i, _):
        plsc.store_scatter(hist, [iota, zero16i + i], zero16f)
        return 0
    lax.fori_loop(0, NB, init_hist, 0)

    def scan(buf, cursor):
        def grp(j, cur):
            srcv = plsc.load_gather(buf, [zero16i, j * 16 + iota])
            dstv = plsc.load_gather(buf, [zero16i + 1, j * 16 + iota])
            ldst = dstv - lo
            m = (ldst >= 0) & (ldst < NB)
            mi = jnp.where(m, 1, 0).astype(jnp.int32)
            pos = plsc.cumsum(mi) - mi
            idx = cur + pos
            plsc.store_scatter(srco, [idx], srcv, mask=m)
            plsc.store_scatter(ldsto, [idx], ldst, mask=m)
            plsc.addupdate_scatter(hist, [iota, ldst], zero16f + 1.0, mask=m)
            return cur + jnp.sum(mi)
        return plsc.parallel_loop(0, CHS // 16, carry=cursor)(grp)

    start(0, gb0, sem0)

    def pair(p, cursor):
        start(2 * p + 1, gb1, sem1)
        wait(gb0, sem0)
        cursor = scan(gb0, cursor)
        start(2 * p + 2, gb0, sem0)
        wait(gb1, sem1)
        return scan(gb1, cursor)
    cursor = lax.fori_loop(0, NC_SETUP // 2 - 1, pair, jnp.int32(0))

    start(NC_SETUP - 1, gb1, sem1)
    wait(gb0, sem0)
    cursor = scan(gb0, cursor)
    wait(gb1, sem1)
    cursor = scan(gb1, cursor)

    cpad = ((cursor + (2 * CH - 1)) // (2 * CH)) * (2 * CH)
    cpad = jnp.maximum(cpad, 2 * CH)

    plsc.store_scatter(cntb, [iota], zero16i + cpad)

    def dred(rb, run):
        base = rb * 16 + iota
        acc = zero16f
        for j in range(16):
            acc = acc + plsc.load_gather(hist, [zero16i + j, base])
        plsc.store_scatter(degb, [base], acc)
        di = acc.astype(jnp.int32)
        cs = plsc.cumsum(di)
        plsc.store_scatter(offs, [base], run + cs - di)
        return run + jnp.max(cs)
    lax.fori_loop(0, NB // 16, dred, jnp.int32(0))
    plsc.store_scatter(offs, [NB + iota], zero16i + cursor)

    def sortg(j, _):
        idx = j * 16 + iota
        ldv = plsc.load_gather(ldsto, [idx])
        sv = plsc.load_gather(srco, [idx])
        cntv, lastm = plsc.scan_count(ldv)
        basev = plsc.load_gather(offs, [ldv])
        pos = basev + cntv - _CNT_BASE
        plsc.store_scatter(srcs, [pos], sv)
        plsc.store_scatter(ldsts, [pos], ldv)
        plsc.addupdate_scatter(offs, [ldv], cntv + (1 - _CNT_BASE),
                               mask=lastm)
        return 0
    lax.fori_loop(0, cpad // 16, sortg, 0)

    pltpu.sync_copy(srcs, srcl_hbm.at[wid])
    pltpu.sync_copy(ldsts, ldstl_hbm.at[wid])
    pltpu.sync_copy(cntb, cnt_hbm.at[wid])
    pltpu.sync_copy(degb, deg_hbm.at[pl.ds(lo, NB)])


@functools.partial(
    pl.kernel,
    out_type=(
        jax.ShapeDtypeStruct((NPAD, D), jnp.float32),
        jax.ShapeDtypeStruct((NPAD, D), jnp.float32),
        jax.ShapeDtypeStruct((NPAD, D), jnp.float32),
    ),
    mesh=_mesh,
    compiler_params=_sc_params,
    scratch_types=[
        pltpu.VMEM((CAP,), jnp.int32),
        pltpu.VMEM((CAP,), jnp.int32),
        pltpu.VMEM((CH, D), jnp.float32),
        pltpu.VMEM((CH, D), jnp.float32),
    ] + [pltpu.VMEM((NBP, 16), jnp.float32) for _ in range(12)] + [
        pltpu.VMEM((16,), jnp.int32),
        pltpu.SemaphoreType.DMA,
        pltpu.SemaphoreType.DMA,
    ],
)
def _sc_agg(x_hbm, srcl_hbm, ldstl_hbm, cnt_hbm, sum_hbm, max_hbm, min_hbm,
            srcb, ldstb, rows0, rows1,
            s0, s1, s2, s3, m0, m1, m2, m3, n0, n1, n2, n3,
            cntb, sem0, sem1):
    acc_s = [s0, s1, s2, s3]
    acc_m = [m0, m1, m2, m3]
    acc_n = [n0, n1, n2, n3]
    cid = lax.axis_index("c")
    sid = lax.axis_index("s")
    wid = sid * 2 + cid
    iota = lax.iota(jnp.int32, 16)
    zero16i = jnp.zeros((16,), jnp.int32)
    zero16f = jnp.zeros((16,), jnp.float32)

    pltpu.sync_copy(srcl_hbm.at[wid], srcb)
    pltpu.sync_copy(ldstl_hbm.at[wid], ldstb)
    pltpu.sync_copy(cnt_hbm.at[wid], cntb)
    cnt = jnp.max(plsc.load_gather(cntb, [iota]))

    def start(ci, buf, sem):
        pltpu.async_copy(x_hbm.at[srcb.at[pl.ds(ci * CH, CH)]], buf, sem)

    def wait(buf, sem):
        pltpu.make_async_copy(x_hbm.at[srcb.at[pl.ds(0, CH)]], buf, sem).wait()

    def ibody(r, _):
        rr = zero16i + r
        for k in range(D // 16):
            plsc.store_scatter(acc_s[k], [rr, iota], zero16f)
            plsc.store_scatter(acc_m[k], [rr, iota], zero16f + _NEG)
            plsc.store_scatter(acc_n[k], [rr, iota], zero16f + _POS)
        return 0
    lax.fori_loop(0, NBP, ibody, 0)

    neutral = ((zero16f,) * 4, (zero16f + _NEG,) * 4, (zero16f + _POS,) * 4)

    def flush(rs, rm, rn, curs):
        curv = zero16i + curs
        for k in range(D // 16):
            plsc.store_scatter(acc_s[k], [curv, iota], rs[k])
            plsc.store_scatter(acc_m[k], [curv, iota], rm[k])
            plsc.store_scatter(acc_n[k], [curv, iota], rn[k])

    def proc(rows, ci, state):
        base = ci * CH

        def grp(j, st):
            rs, rm, rn, curs = st
            dvv = ldstb[pl.ds(base + j * 16, 16)]
            for e in range(16):
                dsc = dvv[e]

                def do_flush(rs, rm, rn, curs):
                    flush(rs, rm, rn, curs)
                    return neutral + (dsc,)

                def no_flush(rs, rm, rn, curs):
                    return (rs, rm, rn, curs)

                rs, rm, rn, curs = lax.cond(dsc != curs, do_flush, no_flush,
                                            rs, rm, rn, curs)
                er = zero16i + (j * 16 + e)
                mvs = [plsc.load_gather(rows, [er, k * 16 + iota])
                       for k in range(D // 16)]
                rs = tuple(rs[k] + mvs[k] for k in range(D // 16))
                rm = tuple(jnp.maximum(rm[k], mvs[k])
                           for k in range(D // 16))
                rn = tuple(jnp.minimum(rn[k], mvs[k])
                           for k in range(D // 16))
            return (rs, rm, rn, curs)
        return lax.fori_loop(0, CH // 16, grp, state)

    nc = cnt // CH
    state = neutral + (jnp.int32(NB),)
    start(0, rows0, sem0)

    def pair(p, st):
        start(2 * p + 1, rows1, sem1)
        wait(rows0, sem0)
        st = proc(rows0, 2 * p, st)
        start(2 * p + 2, rows0, sem0)
        wait(rows1, sem1)
        return proc(rows1, 2 * p + 1, st)
    state = lax.fori_loop(0, nc // 2 - 1, pair, state)

    start(nc - 1, rows1, sem1)
    wait(rows0, sem0)
    state = proc(rows0, nc - 2, state)
    wait(rows1, sem1)
    state = proc(rows1, nc - 1, state)
    flush(*state)

    for k in range(D // 16):
        ks = pl.ds(k * 16, 16)
        pltpu.sync_copy(acc_s[k].at[pl.ds(0, NB)],
                        sum_hbm.at[pl.ds(wid * NB, NB), ks])
        pltpu.sync_copy(acc_m[k].at[pl.ds(0, NB)],
                        max_hbm.at[pl.ds(wid * NB, NB), ks])
        pltpu.sync_copy(acc_n[k].at[pl.ds(0, NB)],
                        min_hbm.at[pl.ds(wid * NB, NB), ks])


_BR = 2048


def _embed_body(h_ref, w_ref, b_ref, o_ref):
    o_ref[...] = jnp.dot(h_ref[...], w_ref[...],
                         preferred_element_type=jnp.float32) + b_ref[...]


def _embed(hpad, w, b):
    return pl.pallas_call(
        _embed_body,
        out_shape=jax.ShapeDtypeStruct((NPAD, D), jnp.float32),
        grid=(NPAD // _BR,),
        in_specs=[
            pl.BlockSpec((_BR, IN_DIM), lambda i: (i, 0)),
            pl.BlockSpec((IN_DIM, D), lambda i: (0, 0)),
            pl.BlockSpec((1, D), lambda i: (0, 0)),
        ],
        out_specs=pl.BlockSpec((_BR, D), lambda i: (i, 0)),
    )(hpad, w, b)


def _combine_body(x_ref, s_ref, mx_ref, mn_ref, deg_ref, sn_ref, w_ref, b_ref,
                  o_ref):
    deg = deg_ref[...]
    x = x_ref[...]
    pos = deg > 0.0
    mean = s_ref[...] / jnp.maximum(deg, 1.0)
    mx = jnp.where(pos, mx_ref[...], 0.0)
    mn = jnp.where(pos, mn_ref[...], 0.0)
    la = jnp.log(deg + 1.0)
    amp = la / AVG_LOG
    att = jnp.where(la > 0.0, AVG_LOG / jnp.maximum(la, 1e-6), 1.0)
    cat = jnp.concatenate(
        [x, mean, mean * amp, mean * att, mx, mx * amp, mx * att,
         mn, mn * amp, mn * att], axis=1)
    out = jnp.dot(cat, w_ref[...], preferred_element_type=jnp.float32)
    out = out + b_ref[...]
    out = jnp.maximum(out * sn_ref[...], 0.0) + x
    o_ref[...] = out


def _combine(x, s, mx, mn, deg, sn, w, b):
    return pl.pallas_call(
        _combine_body,
        out_shape=jax.ShapeDtypeStruct((NPAD, D), jnp.float32),
        grid=(NPAD // _BR,),
        in_specs=[
            pl.BlockSpec((_BR, D), lambda i: (i, 0)),
            pl.BlockSpec((_BR, D), lambda i: (i, 0)),
            pl.BlockSpec((_BR, D), lambda i: (i, 0)),
            pl.BlockSpec((_BR, D), lambda i: (i, 0)),
            pl.BlockSpec((_BR, 1), lambda i: (i, 0)),
            pl.BlockSpec((_BR, 1), lambda i: (i, 0)),
            pl.BlockSpec((CAT_DIM, D), lambda i: (0, 0)),
            pl.BlockSpec((1, D), lambda i: (0, 0)),
        ],
        out_specs=pl.BlockSpec((_BR, D), lambda i: (i, 0)),
    )(x, s, mx, mn, deg, sn, w, b)


def _readout_body(x_ref, w0_ref, b0_ref, w1_ref, b1_ref, w2_ref, b2_ref,
                  o_ref):
    y = jnp.dot(x_ref[...], w0_ref[...], preferred_element_type=jnp.float32)
    y = jnp.maximum(y + b0_ref[...], 0.0)
    y = jnp.dot(y, w1_ref[...], preferred_element_type=jnp.float32)
    y = jnp.maximum(y + b1_ref[...], 0.0)
    y = jnp.dot(y, w2_ref[...], preferred_element_type=jnp.float32)
    o_ref[...] = y + b2_ref[...]


def _readout(x, w0, b0, w1, b1, w2, b2):
    return pl.pallas_call(
        _readout_body,
        out_shape=jax.ShapeDtypeStruct((NPAD, 1), jnp.float32),
        grid=(NPAD // _BR,),
        in_specs=[
            pl.BlockSpec((_BR, D), lambda i: (i, 0)),
            pl.BlockSpec((D, 32), lambda i: (0, 0)),
            pl.BlockSpec((1, 32), lambda i: (0, 0)),
            pl.BlockSpec((32, 16), lambda i: (0, 0)),
            pl.BlockSpec((1, 16), lambda i: (0, 0)),
            pl.BlockSpec((16, 1), lambda i: (0, 0)),
            pl.BlockSpec((1, 1), lambda i: (0, 0)),
        ],
        out_specs=pl.BlockSpec((_BR, 1), lambda i: (i, 0)),
    )(x, w0, b0, w1, b1, w2, b2)


def kernel(g, h, e, snorm_n, snorm_e, W_emb, b_emb, W_l0, b_l0, W_l1, b_l1,
           W_l2, b_l2, W_l3, b_l3, W_r0, b_r0, W_r1, b_r1, W_r2, b_r2):
    srcl, ldstl, cnts, deg = _sc_setup(g)
    deg2 = deg[:, None]
    hpad = jnp.pad(h, ((0, NPAD - N), (0, 0)))
    snp = jnp.pad(snorm_n, ((0, NPAD - N), (0, 0)))
    x = _embed(hpad, W_emb, b_emb[None, :])
    for W, b in ((W_l0, b_l0), (W_l1, b_l1), (W_l2, b_l2), (W_l3, b_l3)):
        s, mx, mn = _sc_agg(x, srcl, ldstl, cnts)
        x = _combine(x, s, mx, mn, deg2, snp, W, b[None, :])
    y = _readout(x, W_r0, b_r0[None, :], W_r1, b_r1[None, :],
                 W_r2, b_r2[None, :])
    return y[:N]


# speedup vs baseline: 2.5037x; 1.0954x over previous
import functools

import jax
import jax.numpy as jnp
from jax import lax
from jax.experimental import pallas as pl
from jax.experimental.pallas import tpu as pltpu
from jax.experimental.pallas import tpu_sc as plsc

N = 10000
E = 320000
IN_DIM = 128
D = 64
AVG_LOG = 3.4965075614664802
CAT_DIM = 10 * D

NT = 32
NB = 320
NPAD = NT * NB
NBP = NB + 8
CAP = 12288
CH = 128
CHS = 2000

_NEG = -3.0e38
_POS = 3.0e38
_CNT_BASE = 1

_mesh = plsc.VectorSubcoreMesh(core_axis_name="c", subcore_axis_name="s")
_sc_params = pltpu.CompilerParams(needs_layout_passes=False,
                                  use_tc_tiling_on_sc=False)


def _lane_bcast(v, j):
    idx = jnp.full((16, 1), j, jnp.int32)
    return lax.gather(
        v, idx,
        lax.GatherDimensionNumbers(
            offset_dims=(), collapsed_slice_dims=(0,), start_index_map=(0,)),
        (1,), mode=lax.GatherScatterMode.PROMISE_IN_BOUNDS)


NC_SETUP = E // CHS


@functools.partial(
    pl.kernel,
    out_type=(
        jax.ShapeDtypeStruct((NT, CAP), jnp.int32),
        jax.ShapeDtypeStruct((NT, CAP), jnp.int32),
        jax.ShapeDtypeStruct((NT, 16), jnp.int32),
        jax.ShapeDtypeStruct((NPAD,), jnp.float32),
    ),
    mesh=_mesh,
    compiler_params=_sc_params,
    scratch_types=[
        pltpu.VMEM((2, CHS), jnp.int32),
        pltpu.VMEM((2, CHS), jnp.int32),
        pltpu.VMEM((CAP,), jnp.int32),
        pltpu.VMEM((CAP,), jnp.int32),
        pltpu.VMEM((16, NB), jnp.float32),
        pltpu.VMEM((NB,), jnp.float32),
        pltpu.VMEM((16,), jnp.int32),
        pltpu.VMEM((NB + 16,), jnp.int32),
        pltpu.VMEM((CAP,), jnp.int32),
        pltpu.VMEM((CAP,), jnp.int32),
        pltpu.SemaphoreType.DMA,
        pltpu.SemaphoreType.DMA,
    ],
)
def _sc_setup(g_hbm, srcl_hbm, ldstl_hbm, cnt_hbm, deg_hbm,
              gb0, gb1, srco, ldsto, hist, degb, cntb, offs, srcs, ldsts,
              sem0, sem1):
    cid = lax.axis_index("c")
    sid = lax.axis_index("s")
    wid = sid * 2 + cid
    lo = wid * NB
    iota = lax.iota(jnp.int32, 16)
    zero16f = jnp.zeros((16,), jnp.float32)
    zero16i = jnp.zeros((16,), jnp.int32)

    def start(ci, buf, sem):
        pltpu.async_copy(g_hbm.at[:, pl.ds(ci * CHS, CHS)], buf, sem)

    def wait(buf, sem):
        pltpu.make_async_copy(g_hbm.at[:, pl.ds(0, CHS)], buf, sem).wait()

    def init_lists(i, _):
        idx = i * 16 + iota
        plsc.store_scatter(srco, [idx], zero16i)
        plsc.store_scatter(ldsto, [idx], zero16i + NB)
        return 0
    lax.fori_loop(0, CAP // 16, init_lists, 0)

    def init_hist(i, _):
        plsc.store_scatter(hist, [iota, zero16i + i], zero16f)
        return 0
    lax.fori_loop(0, NB, init_hist, 0)

    def scan(buf, cursor):
        def grp(j, cur):
            srcv = plsc.load_gather(buf, [zero16i, j * 16 + iota])
            dstv = plsc.load_gather(buf, [zero16i + 1, j * 16 + iota])
            ldst = dstv - lo
            m = (ldst >= 0) & (ldst < NB)
            mi = jnp.where(m, 1, 0).astype(jnp.int32)
            pos = plsc.cumsum(mi) - mi
            idx = cur + pos
            plsc.store_scatter(srco, [idx], srcv, mask=m)
            plsc.store_scatter(ldsto, [idx], ldst, mask=m)
            plsc.addupdate_scatter(hist, [iota, ldst], zero16f + 1.0, mask=m)
            return cur + jnp.sum(mi)
        return plsc.parallel_loop(0, CHS // 16, carry=cursor)(grp)

    start(0, gb0, sem0)

    def pair(p, cursor):
        start(2 * p + 1, gb1, sem1)
        wait(gb0, sem0)
        cursor = scan(gb0, cursor)
        start(2 * p + 2, gb0, sem0)
        wait(gb1, sem1)
        return scan(gb1, cursor)
    cursor = lax.fori_loop(0, NC_SETUP // 2 - 1, pair, jnp.int32(0))

    start(NC_SETUP - 1, gb1, sem1)
    wait(gb0, sem0)
    cursor = scan(gb0, cursor)
    wait(gb1, sem1)
    cursor = scan(gb1, cursor)

    cpad = ((cursor + (2 * CH - 1)) // (2 * CH)) * (2 * CH)
    cpad = jnp.maximum(cpad, 2 * CH)

    plsc.store_scatter(cntb, [iota], zero16i + cpad)

    def dred(rb, run):
        base = rb * 16 + iota
        acc = zero16f
        for j in range(16):
            acc = acc + plsc.load_gather(hist, [zero16i + j, base])
        plsc.store_scatter(degb, [base], acc)
        di = acc.astype(jnp.int32)
        cs = plsc.cumsum(di)
        plsc.store_scatter(offs, [base], run + cs - di)
        return run + jnp.max(cs)
    lax.fori_loop(0, NB // 16, dred, jnp.int32(0))
    plsc.store_scatter(offs, [NB + iota], zero16i + cursor)

    def sortg(j, _):
        idx = j * 16 + iota
        ldv = plsc.load_gather(ldsto, [idx])
        sv = plsc.load_gather(srco, [idx])
        cntv, lastm = plsc.scan_count(ldv)
        basev = plsc.load_gather(offs, [ldv])
        pos = basev + cntv - _CNT_BASE
        plsc.store_scatter(srcs, [pos], sv)
        plsc.store_scatter(ldsts, [pos], ldv)
        plsc.addupdate_scatter(offs, [ldv], cntv + (1 - _CNT_BASE),
                               mask=lastm)
        return 0
    lax.fori_loop(0, cpad // 16, sortg, 0)

    pltpu.sync_copy(srcs, srcl_hbm.at[wid])
    pltpu.sync_copy(ldsts, ldstl_hbm.at[wid])
    pltpu.sync_copy(cntb, cnt_hbm.at[wid])
    pltpu.sync_copy(degb, deg_hbm.at[pl.ds(lo, NB)])


@functools.partial(
    pl.kernel,
    out_type=(
        jax.ShapeDtypeStruct((NPAD, D), jnp.float32),
        jax.ShapeDtypeStruct((NPAD, D), jnp.float32),
        jax.ShapeDtypeStruct((NPAD, D), jnp.float32),
    ),
    mesh=_mesh,
    compiler_params=_sc_params,
    scratch_types=[
        pltpu.VMEM((CAP,), jnp.int32),
        pltpu.VMEM((CAP,), jnp.int32),
        pltpu.VMEM((CH, D), jnp.float32),
        pltpu.VMEM((CH, D), jnp.float32),
    ] + [pltpu.VMEM((NBP, 16), jnp.float32) for _ in range(12)] + [
        pltpu.VMEM((16,), jnp.int32),
        pltpu.SemaphoreType.DMA,
        pltpu.SemaphoreType.DMA,
    ],
)
def _sc_agg(x_hbm, srcl_hbm, ldstl_hbm, cnt_hbm, sum_hbm, max_hbm, min_hbm,
            srcb, ldstb, rows0, rows1,
            s0, s1, s2, s3, m0, m1, m2, m3, n0, n1, n2, n3,
            cntb, sem0, sem1):
    acc_s = [s0, s1, s2, s3]
    acc_m = [m0, m1, m2, m3]
    acc_n = [n0, n1, n2, n3]
    cid = lax.axis_index("c")
    sid = lax.axis_index("s")
    wid = sid * 2 + cid
    iota = lax.iota(jnp.int32, 16)
    zero16i = jnp.zeros((16,), jnp.int32)
    zero16f = jnp.zeros((16,), jnp.float32)

    pltpu.sync_copy(srcl_hbm.at[wid], srcb)
    pltpu.sync_copy(ldstl_hbm.at[wid], ldstb)
    pltpu.sync_copy(cnt_hbm.at[wid], cntb)
    cnt = jnp.max(plsc.load_gather(cntb, [iota]))

    def start(ci, buf, sem):
        pltpu.async_copy(x_hbm.at[srcb.at[pl.ds(ci * CH, CH)]], buf, sem)

    def wait(buf, sem):
        pltpu.make_async_copy(x_hbm.at[srcb.at[pl.ds(0, CH)]], buf, sem).wait()

    def ibody(r, _):
        rr = zero16i + r
        for k in range(D // 16):
            plsc.store_scatter(acc_s[k], [rr, iota], zero16f)
            plsc.store_scatter(acc_m[k], [rr, iota], zero16f + _NEG)
            plsc.store_scatter(acc_n[k], [rr, iota], zero16f + _POS)
        return 0
    lax.fori_loop(0, NBP, ibody, 0)

    neutral = ((zero16f,) * 4, (zero16f + _NEG,) * 4, (zero16f + _POS,) * 4)

    def flush(rs, rm, rn, curs):
        curv = zero16i + curs
        for k in range(D // 16):
            plsc.store_scatter(acc_s[k], [curv, iota], rs[k])
            plsc.store_scatter(acc_m[k], [curv, iota], rm[k])
            plsc.store_scatter(acc_n[k], [curv, iota], rn[k])

    def proc(rows, ci, state):
        base = ci * CH

        def grp(j, st):
            dvv = ldstb[pl.ds(base + j * 16, 16)]

            def accum(e, rs, rm, rn):
                er = zero16i + (j * 16 + e)
                mvs = [plsc.load_gather(rows, [er, k * 16 + iota])
                       for k in range(D // 16)]
                rs = tuple(rs[k] + mvs[k] for k in range(D // 16))
                rm = tuple(jnp.maximum(rm[k], mvs[k])
                           for k in range(D // 16))
                rn = tuple(jnp.minimum(rn[k], mvs[k])
                           for k in range(D // 16))
                return rs, rm, rn

            def fast(rs, rm, rn, curs):
                for e in range(16):
                    rs, rm, rn = accum(e, rs, rm, rn)
                return (rs, rm, rn, curs)

            def slow(rs, rm, rn, curs):
                for e in range(16):
                    dsc = dvv[e]

                    def do_flush(rs, rm, rn, curs):
                        flush(rs, rm, rn, curs)
                        return neutral + (dsc,)

                    def no_flush(rs, rm, rn, curs):
                        return (rs, rm, rn, curs)

                    rs, rm, rn, curs = lax.cond(dsc != curs, do_flush,
                                                no_flush, rs, rm, rn, curs)
                    rs, rm, rn = accum(e, rs, rm, rn)
                return (rs, rm, rn, curs)

            rs, rm, rn, curs = st
            uniform = (jnp.max(dvv) == curs) & (jnp.min(dvv) == curs)
            return lax.cond(uniform, fast, slow, rs, rm, rn, curs)
        return lax.fori_loop(0, CH // 16, grp, state)

    nc = cnt // CH
    state = neutral + (jnp.int32(NB),)
    start(0, rows0, sem0)

    def pair(p, st):
        start(2 * p + 1, rows1, sem1)
        wait(rows0, sem0)
        st = proc(rows0, 2 * p, st)
        start(2 * p + 2, rows0, sem0)
        wait(rows1, sem1)
        return proc(rows1, 2 * p + 1, st)
    state = lax.fori_loop(0, nc // 2 - 1, pair, state)

    start(nc - 1, rows1, sem1)
    wait(rows0, sem0)
    state = proc(rows0, nc - 2, state)
    wait(rows1, sem1)
    state = proc(rows1, nc - 1, state)
    flush(*state)

    for k in range(D // 16):
        ks = pl.ds(k * 16, 16)
        pltpu.sync_copy(acc_s[k].at[pl.ds(0, NB)],
                        sum_hbm.at[pl.ds(wid * NB, NB), ks])
        pltpu.sync_copy(acc_m[k].at[pl.ds(0, NB)],
                        max_hbm.at[pl.ds(wid * NB, NB), ks])
        pltpu.sync_copy(acc_n[k].at[pl.ds(0, NB)],
                        min_hbm.at[pl.ds(wid * NB, NB), ks])


_BR = 2048


def _embed_body(h_ref, w_ref, b_ref, o_ref):
    o_ref[...] = jnp.dot(h_ref[...], w_ref[...],
                         preferred_element_type=jnp.float32) + b_ref[...]


def _embed(hpad, w, b):
    return pl.pallas_call(
        _embed_body,
        out_shape=jax.ShapeDtypeStruct((NPAD, D), jnp.float32),
        grid=(NPAD // _BR,),
        in_specs=[
            pl.BlockSpec((_BR, IN_DIM), lambda i: (i, 0)),
            pl.BlockSpec((IN_DIM, D), lambda i: (0, 0)),
            pl.BlockSpec((1, D), lambda i: (0, 0)),
        ],
        out_specs=pl.BlockSpec((_BR, D), lambda i: (i, 0)),
    )(hpad, w, b)


def _combine_body(x_ref, s_ref, mx_ref, mn_ref, deg_ref, sn_ref, w_ref, b_ref,
                  o_ref):
    deg = deg_ref[...]
    x = x_ref[...]
    pos = deg > 0.0
    mean = s_ref[...] / jnp.maximum(deg, 1.0)
    mx = jnp.where(pos, mx_ref[...], 0.0)
    mn = jnp.where(pos, mn_ref[...], 0.0)
    la = jnp.log(deg + 1.0)
    amp = la / AVG_LOG
    att = jnp.where(la > 0.0, AVG_LOG / jnp.maximum(la, 1e-6), 1.0)
    cat = jnp.concatenate(
        [x, mean, mean * amp, mean * att, mx, mx * amp, mx * att,
         mn, mn * amp, mn * att], axis=1)
    out = jnp.dot(cat, w_ref[...], preferred_element_type=jnp.float32)
    out = out + b_ref[...]
    out = jnp.maximum(out * sn_ref[...], 0.0) + x
    o_ref[...] = out


def _combine(x, s, mx, mn, deg, sn, w, b):
    return pl.pallas_call(
        _combine_body,
        out_shape=jax.ShapeDtypeStruct((NPAD, D), jnp.float32),
        grid=(NPAD // _BR,),
        in_specs=[
            pl.BlockSpec((_BR, D), lambda i: (i, 0)),
            pl.BlockSpec((_BR, D), lambda i: (i, 0)),
            pl.BlockSpec((_BR, D), lambda i: (i, 0)),
            pl.BlockSpec((_BR, D), lambda i: (i, 0)),
            pl.BlockSpec((_BR, 1), lambda i: (i, 0)),
            pl.BlockSpec((_BR, 1), lambda i: (i, 0)),
            pl.BlockSpec((CAT_DIM, D), lambda i: (0, 0)),
            pl.BlockSpec((1, D), lambda i: (0, 0)),
        ],
        out_specs=pl.BlockSpec((_BR, D), lambda i: (i, 0)),
    )(x, s, mx, mn, deg, sn, w, b)


def _readout_body(x_ref, w0_ref, b0_ref, w1_ref, b1_ref, w2_ref, b2_ref,
                  o_ref):
    y = jnp.dot(x_ref[...], w0_ref[...], preferred_element_type=jnp.float32)
    y = jnp.maximum(y + b0_ref[...], 0.0)
    y = jnp.dot(y, w1_ref[...], preferred_element_type=jnp.float32)
    y = jnp.maximum(y + b1_ref[...], 0.0)
    y = jnp.dot(y, w2_ref[...], preferred_element_type=jnp.float32)
    o_ref[...] = y + b2_ref[...]


def _readout(x, w0, b0, w1, b1, w2, b2):
    return pl.pallas_call(
        _readout_body,
        out_shape=jax.ShapeDtypeStruct((NPAD, 1), jnp.float32),
        grid=(NPAD // _BR,),
        in_specs=[
            pl.BlockSpec((_BR, D), lambda i: (i, 0)),
            pl.BlockSpec((D, 32), lambda i: (0, 0)),
            pl.BlockSpec((1, 32), lambda i: (0, 0)),
            pl.BlockSpec((32, 16), lambda i: (0, 0)),
            pl.BlockSpec((1, 16), lambda i: (0, 0)),
            pl.BlockSpec((16, 1), lambda i: (0, 0)),
            pl.BlockSpec((1, 1), lambda i: (0, 0)),
        ],
        out_specs=pl.BlockSpec((_BR, 1), lambda i: (i, 0)),
    )(x, w0, b0, w1, b1, w2, b2)


def kernel(g, h, e, snorm_n, snorm_e, W_emb, b_emb, W_l0, b_l0, W_l1, b_l1,
           W_l2, b_l2, W_l3, b_l3, W_r0, b_r0, W_r1, b_r1, W_r2, b_r2):
    srcl, ldstl, cnts, deg = _sc_setup(g)
    deg2 = deg[:, None]
    hpad = jnp.pad(h, ((0, NPAD - N), (0, 0)))
    snp = jnp.pad(snorm_n, ((0, NPAD - N), (0, 0)))
    x = _embed(hpad, W_emb, b_emb[None, :])
    for W, b in ((W_l0, b_l0), (W_l1, b_l1), (W_l2, b_l2), (W_l3, b_l3)):
        s, mx, mn = _sc_agg(x, srcl, ldstl, cnts)
        x = _combine(x, s, mx, mn, deg2, snp, W, b[None, :])
    y = _readout(x, W_r0, b_r0[None, :], W_r1, b_r1[None, :],
                 W_r2, b_r2[None, :])
    return y[:N]
